# baseline (speedup 1.0000x reference)
import jax
import jax.numpy as jnp
from jax.experimental import pallas as pl
from jax.experimental.pallas import tpu as pltpu

TB = 256


def _router_body(x_ref, w_ref, pi_ref, b_ref, out_ref):
    nb = x_ref.shape[0]
    w = w_ref[...]
    scale = pi_ref[...]
    bias = b_ref[...]
    for bi in range(nb):
        scores = jax.lax.dot_general(
            x_ref[bi], w, (((1,), (1,)), ((), ())),
            preferred_element_type=jnp.float32,
        )
        out_ref[:, bi, :] = (scores + bias) * scale


def kernel(seq, pi, W, b):
    B, T, D = seq.shape
    E = W.shape[0]
    grid = (T // TB,)
    full = pl.pallas_call(
        _router_body,
        grid=grid,
        in_specs=[
            pl.BlockSpec((B, TB, D), lambda tc: (0, tc, 0)),
            pl.BlockSpec((E, D), lambda tc: (0, 0)),
            pl.BlockSpec((1, E), lambda tc: (0, 0)),
            pl.BlockSpec((1, E), lambda tc: (0, 0)),
        ],
        out_specs=pl.BlockSpec((TB, B, E), lambda tc: (tc, 0, 0)),
        out_shape=jax.ShapeDtypeStruct((T, B, E), jnp.float32),
    )(seq, W, pi.reshape(1, E), b.reshape(1, E))
    return full[1:]
